# baseline (speedup 1.0000x reference)
import jax
import jax.numpy as jnp
from jax import lax
from jax.experimental import pallas as pl
from jax.experimental.pallas import tpu as pltpu
from jax.experimental.pallas import tpu_sc as plsc

VOCAB = 100000
EMBED_DIM = 128
BATCH = 4096
SEQ = 50

_INFO = plsc.get_sparse_core_info()
NC = _INFO.num_cores
NS = _INFO.num_subcores
NW = NC * NS

SAMPLES_PER_W = BATCH // NW
NCHUNK = SEQ
NBUF = 5
LAG = 2


def _body(senA_hbm, senB_hbm, table_hbm, outA_hbm, outB_hbm,
          idx_v, rows_bufs, gsems, ssems):
    wid = lax.axis_index("s") * NC + lax.axis_index("c")
    sbase = wid * SAMPLES_PER_W

    for sen_hbm, out_hbm in ((senA_hbm, outA_hbm), (senB_hbm, outB_hbm)):
        pltpu.sync_copy(sen_hbm.at[wid], idx_v)

        for b in range(NBUF):
            pltpu.async_copy(table_hbm.at[idx_v.at[b]], rows_bufs[b], gsems[b])

        def round_(t0, _):
            for b in range(NBUF):
                t = t0 + b
                pltpu.make_async_copy(table_hbm.at[idx_v.at[t]], rows_bufs[b],
                                      gsems[b]).wait()
                pltpu.async_copy(
                    rows_bufs[b], out_hbm.at[t, pl.ds(sbase, SAMPLES_PER_W)],
                    ssems[b])

                b2 = (b - LAG) % NBUF
                t_new = t - LAG + NBUF

                @pl.when(jnp.logical_and(t >= LAG, t_new < NCHUNK))
                def _():
                    pltpu.make_async_copy(
                        rows_bufs[b2],
                        out_hbm.at[t - LAG, pl.ds(sbase, SAMPLES_PER_W)],
                        ssems[b2]).wait()
                    pltpu.async_copy(table_hbm.at[idx_v.at[t_new]],
                                     rows_bufs[b2], gsems[b2])
            return _

        lax.fori_loop(0, NCHUNK // NBUF, lambda i, c: round_(i * NBUF, c),
                      None)

        for b in range(NBUF):
            t = NCHUNK - NBUF + b
            pltpu.make_async_copy(
                rows_bufs[b], out_hbm.at[t, pl.ds(sbase, SAMPLES_PER_W)],
                ssems[b]).wait()


@jax.jit
def _gather_all(senA3, senB3, table):
    mesh = plsc.VectorSubcoreMesh(core_axis_name="c", subcore_axis_name="s")
    kern = pl.kernel(
        _body,
        out_type=(
            jax.ShapeDtypeStruct((SEQ, BATCH, EMBED_DIM), jnp.float32),
            jax.ShapeDtypeStruct((SEQ, BATCH, EMBED_DIM), jnp.float32),
        ),
        mesh=mesh,
        scratch_types=[
            pltpu.VMEM((NCHUNK, SAMPLES_PER_W), jnp.int32),
            [pltpu.VMEM((SAMPLES_PER_W, EMBED_DIM), jnp.float32)
             for _ in range(NBUF)],
            [pltpu.SemaphoreType.DMA for _ in range(NBUF)],
            [pltpu.SemaphoreType.DMA for _ in range(NBUF)],
        ],
    )
    return kern(senA3, senB3, table)


def kernel(senA, senB, table):
    senA3 = senA.T.reshape(SEQ, NW, SAMPLES_PER_W).transpose(1, 0, 2)
    senB3 = senB.T.reshape(SEQ, NW, SAMPLES_PER_W).transpose(1, 0, 2)
    outA3, outB3 = _gather_all(senA3, senB3, table)
    return outA3.transpose(1, 0, 2), outB3.transpose(1, 0, 2)
